# Initial kernel scaffold; baseline (speedup 1.0000x reference)
import jax
import jax.numpy as jnp
from jax import lax
from jax.experimental import pallas as pl
from jax.experimental.pallas import tpu as pltpu
from jax.experimental.pallas import tpu_sc as plsc

_NC = 2
_NS = 16
_NW = _NC * _NS

_BATCH = 16384
_B_PER_W = _BATCH // _NW
_CHUNK = 128
_NCHUNK = _B_PER_W // _CHUNK


def _gather_pipeline(tbl_hbm, out_hbm, idx_v, bufs, gsem, osem, base):
    gathers = [None, None]
    outs = [None, None]
    gathers[0] = pltpu.async_copy(tbl_hbm.at[idx_v.at[0]], bufs.at[0], gsem)
    for j in range(_NCHUNK):
        p = j % 2
        q = (j + 1) % 2
        if j + 1 < _NCHUNK:
            if outs[q] is not None:
                outs[q].wait()
            gathers[q] = pltpu.async_copy(
                tbl_hbm.at[idx_v.at[j + 1]], bufs.at[q], gsem)
        gathers[p].wait()
        outs[p] = pltpu.async_copy(
            bufs.at[p], out_hbm.at[pl.ds(base + j * _CHUNK, _CHUNK)], osem)
    outs[0].wait()
    outs[1].wait()


def _sc_gather(c_hbm, a_hbm, s_hbm, idx_hbm,
               fc_hbm, fa_hbm, fs_hbm,
               idx_v, fc_bufs, wide_bufs, gsem, osem):
    wid = lax.axis_index("s") * _NC + lax.axis_index("c")
    base = wid * _B_PER_W
    for j in range(_NCHUNK):
        pltpu.sync_copy(idx_hbm.at[pl.ds(base + j * _CHUNK, _CHUNK)],
                        idx_v.at[j])
    _gather_pipeline(c_hbm, fc_hbm, idx_v, fc_bufs, gsem, osem, base)
    _gather_pipeline(a_hbm, fa_hbm, idx_v, wide_bufs, gsem, osem, base)
    _gather_pipeline(s_hbm, fs_hbm, idx_v, wide_bufs, gsem, osem, base)


def _fclass_body(fc_ref, w_ref, b_ref, out_ref):
    out_ref[...] = lax.dot_general(
        fc_ref[...], w_ref[...],
        dimension_numbers=(((1,), (1,)), ((), ())),
        preferred_element_type=jnp.float32) + b_ref[...]


@jax.jit
def kernel(c_latent, a_latent, s_latent, W, b, sample_index):
    fc_dim = c_latent.shape[1]
    fa_dim = a_latent.shape[1]
    fs_dim = s_latent.shape[1]
    ncat = W.shape[0]
    idx = sample_index.astype(jnp.int32)

    mesh = plsc.VectorSubcoreMesh(core_axis_name="c", subcore_axis_name="s")
    sc_call = pl.kernel(
        _sc_gather,
        out_type=(
            jax.ShapeDtypeStruct((_BATCH, fc_dim), jnp.float32),
            jax.ShapeDtypeStruct((_BATCH, fa_dim), jnp.float32),
            jax.ShapeDtypeStruct((_BATCH, fs_dim), jnp.float32),
        ),
        mesh=mesh,
        scratch_types=[
            pltpu.VMEM((_NCHUNK, _CHUNK), jnp.int32),
            pltpu.VMEM((2, _CHUNK, fc_dim), jnp.float32),
            pltpu.VMEM((2, _CHUNK, fa_dim), jnp.float32),
            pltpu.SemaphoreType.DMA,
            pltpu.SemaphoreType.DMA,
        ],
    )
    fc, fa, fs = sc_call(c_latent, a_latent, s_latent, idx)

    fclass = pl.pallas_call(
        _fclass_body,
        out_shape=jax.ShapeDtypeStruct((_BATCH, ncat), jnp.float32),
    )(fc, W, b.reshape(1, ncat))

    return (fc, fa, fs, fclass)


# speedup vs baseline: 1.0809x; 1.0809x over previous
import jax
import jax.numpy as jnp
from jax import lax
from jax.experimental import pallas as pl
from jax.experimental.pallas import tpu as pltpu
from jax.experimental.pallas import tpu_sc as plsc

_NC = 2
_NS = 16
_NW = _NC * _NS

_BATCH = 16384
_B_PER_W = _BATCH // _NW
_CHUNK = 64
_NCHUNK = _B_PER_W // _CHUNK
_LANES = 16


def _gather_pipeline(tbl_hbm, out_hbm, idx_v, bufs, gsem, osem, base):
    gathers = [None, None]
    outs = [None, None]
    gathers[0] = pltpu.async_copy(
        tbl_hbm.at[idx_v.at[pl.ds(0, _CHUNK)]], bufs.at[0], gsem)
    for j in range(_NCHUNK):
        p = j % 2
        q = (j + 1) % 2
        if j + 1 < _NCHUNK:
            if outs[q] is not None:
                outs[q].wait()
            gathers[q] = pltpu.async_copy(
                tbl_hbm.at[idx_v.at[pl.ds((j + 1) * _CHUNK, _CHUNK)]],
                bufs.at[q], gsem)
        gathers[p].wait()
        outs[p] = pltpu.async_copy(
            bufs.at[p], out_hbm.at[pl.ds(base + j * _CHUNK, _CHUNK)], osem)
    outs[0].wait()
    outs[1].wait()


def _sc_gather(c_hbm, a_hbm, s_hbm, idx_hbm,
               fc_hbm, fa_hbm, fs_hbm,
               idx_v, fc_buf, wide_bufs, gsem, osem, fcsem):
    wid = lax.axis_index("s") * _NC + lax.axis_index("c")
    base = wid * _B_PER_W
    pltpu.sync_copy(idx_hbm.at[pl.ds(base, _B_PER_W)], idx_v)

    def fc_rows(g, carry):
        vec = idx_v[pl.ds(g * _LANES, _LANES)]
        for l in range(_LANES):
            pltpu.async_copy(c_hbm.at[pl.ds(vec[l], 1)],
                             fc_buf.at[pl.ds(g * _LANES + l, 1)], fcsem)
        return carry

    lax.fori_loop(0, _B_PER_W // _LANES, fc_rows, 0)

    _gather_pipeline(a_hbm, fa_hbm, idx_v, wide_bufs, gsem, osem, base)
    _gather_pipeline(s_hbm, fs_hbm, idx_v, wide_bufs, gsem, osem, base)

    pltpu.make_async_copy(c_hbm.at[pl.ds(0, _B_PER_W)], fc_buf, fcsem).wait()
    pltpu.async_copy(fc_buf, fc_hbm.at[pl.ds(base, _B_PER_W)], osem).wait()


def _fclass_body(fc_ref, w_ref, b_ref, out_ref):
    out_ref[...] = lax.dot_general(
        fc_ref[...], w_ref[...],
        dimension_numbers=(((1,), (1,)), ((), ())),
        preferred_element_type=jnp.float32) + b_ref[...]


@jax.jit
def kernel(c_latent, a_latent, s_latent, W, b, sample_index):
    fc_dim = c_latent.shape[1]
    fa_dim = a_latent.shape[1]
    fs_dim = s_latent.shape[1]
    ncat = W.shape[0]
    idx = sample_index.astype(jnp.int32)

    mesh = plsc.VectorSubcoreMesh(core_axis_name="c", subcore_axis_name="s")
    sc_call = pl.kernel(
        _sc_gather,
        out_type=(
            jax.ShapeDtypeStruct((_BATCH, fc_dim), jnp.float32),
            jax.ShapeDtypeStruct((_BATCH, fa_dim), jnp.float32),
            jax.ShapeDtypeStruct((_BATCH, fs_dim), jnp.float32),
        ),
        mesh=mesh,
        scratch_types=[
            pltpu.VMEM((_B_PER_W,), jnp.int32),
            pltpu.VMEM((_B_PER_W, fc_dim), jnp.float32),
            pltpu.VMEM((2, _CHUNK, fa_dim), jnp.float32),
            pltpu.SemaphoreType.DMA,
            pltpu.SemaphoreType.DMA,
            pltpu.SemaphoreType.DMA,
        ],
    )
    fc, fa, fs = sc_call(c_latent, a_latent, s_latent, idx)

    fclass = pl.pallas_call(
        _fclass_body,
        out_shape=jax.ShapeDtypeStruct((_BATCH, ncat), jnp.float32),
    )(fc, W, b.reshape(1, ncat))

    return (fc, fa, fs, fclass)


# speedup vs baseline: 1.0981x; 1.0159x over previous
import jax
import jax.numpy as jnp
from jax import lax
from jax.experimental import pallas as pl
from jax.experimental.pallas import tpu as pltpu
from jax.experimental.pallas import tpu_sc as plsc

_NC = 2
_NS = 16
_NW = _NC * _NS

_BATCH = 16384
_B_PER_W = _BATCH // _NW
_CHUNK = 64
_NCHUNK = _B_PER_W // _CHUNK
_LANES = 16


_NBUF = 3


def _sc_gather(c_hbm, a_hbm, s_hbm, idx_hbm,
               fc_hbm, fa_hbm, fs_hbm,
               idx_v, fc_buf, wide_bufs, gsem, osem, fcsem):
    wid = lax.axis_index("s") * _NC + lax.axis_index("c")
    base = wid * _B_PER_W
    pltpu.sync_copy(idx_hbm.at[pl.ds(base, _B_PER_W)], idx_v)

    jobs = []
    for j in range(_NCHUNK):
        jobs.append((a_hbm, fa_hbm, j * _CHUNK))
        jobs.append((s_hbm, fs_hbm, j * _CHUNK))

    def start_gather(k, p):
        tbl, _, off = jobs[k]
        return pltpu.async_copy(
            tbl.at[idx_v.at[pl.ds(off, _CHUNK)]], wide_bufs.at[p], gsem)

    gathers = [None] * _NBUF
    outs = [None] * _NBUF
    for k in range(_NBUF):
        gathers[k] = start_gather(k, k)

    def fc_rows(g, carry):
        vec = idx_v[pl.ds(g * _LANES, _LANES)]
        for l in range(_LANES):
            pltpu.async_copy(c_hbm.at[pl.ds(vec[l], 1)],
                             fc_buf.at[pl.ds(g * _LANES + l, 1)], fcsem)
        return carry

    lax.fori_loop(0, _B_PER_W // _LANES, fc_rows, 0)

    njobs = len(jobs)
    for k in range(njobs):
        p = k % _NBUF
        gathers[p].wait()
        _, out_hbm, off = jobs[k]
        outs[p] = pltpu.async_copy(
            wide_bufs.at[p], out_hbm.at[pl.ds(base + off, _CHUNK)], osem)
        if k + _NBUF < njobs:
            outs[p].wait()
            gathers[p] = start_gather(k + _NBUF, p)
            outs[p] = None
    for p in range(_NBUF):
        if outs[p] is not None:
            outs[p].wait()

    pltpu.make_async_copy(c_hbm.at[pl.ds(0, _B_PER_W)], fc_buf, fcsem).wait()
    pltpu.async_copy(fc_buf, fc_hbm.at[pl.ds(base, _B_PER_W)], fcsem).wait()


def _fclass_body(fc_ref, w_ref, b_ref, out_ref):
    out_ref[...] = lax.dot_general(
        fc_ref[...], w_ref[...],
        dimension_numbers=(((1,), (1,)), ((), ())),
        preferred_element_type=jnp.float32) + b_ref[...]


@jax.jit
def kernel(c_latent, a_latent, s_latent, W, b, sample_index):
    fc_dim = c_latent.shape[1]
    fa_dim = a_latent.shape[1]
    fs_dim = s_latent.shape[1]
    ncat = W.shape[0]
    idx = sample_index.astype(jnp.int32)

    mesh = plsc.VectorSubcoreMesh(core_axis_name="c", subcore_axis_name="s")
    sc_call = pl.kernel(
        _sc_gather,
        out_type=(
            jax.ShapeDtypeStruct((_BATCH, fc_dim), jnp.float32),
            jax.ShapeDtypeStruct((_BATCH, fa_dim), jnp.float32),
            jax.ShapeDtypeStruct((_BATCH, fs_dim), jnp.float32),
        ),
        mesh=mesh,
        scratch_types=[
            pltpu.VMEM((_B_PER_W,), jnp.int32),
            pltpu.VMEM((_B_PER_W, fc_dim), jnp.float32),
            pltpu.VMEM((_NBUF, _CHUNK, fa_dim), jnp.float32),
            pltpu.SemaphoreType.DMA,
            pltpu.SemaphoreType.DMA,
            pltpu.SemaphoreType.DMA,
        ],
    )
    fc, fa, fs = sc_call(c_latent, a_latent, s_latent, idx)

    fclass = pl.pallas_call(
        _fclass_body,
        out_shape=jax.ShapeDtypeStruct((_BATCH, ncat), jnp.float32),
    )(fc, W, b.reshape(1, ncat))

    return (fc, fa, fs, fclass)


# speedup vs baseline: 1.4834x; 1.3508x over previous
import jax
import jax.numpy as jnp
from jax import lax
from jax.experimental import pallas as pl
from jax.experimental.pallas import tpu as pltpu
from jax.experimental.pallas import tpu_sc as plsc

_NC = 2
_NS = 16
_NW = _NC * _NS

_BATCH = 16384
_B_PER_W = _BATCH // _NW
_CHUNK = 64
_NCHUNK = _B_PER_W // _CHUNK
_NBUF = 3
_LANES = 16

_NROWS = 100000
_FEATS_PER_W = 2
_FCHUNK = 2048
_NFCHUNK = _BATCH // _FCHUNK


def _fc_phase(c_t_hbm, idx_hbm, fc_t_hbm, wid, sem):
    def body(row_buf, idx_all, out_bufs, osem):
        pltpu.sync_copy(idx_hbm, idx_all)
        for f in range(_FEATS_PER_W):
            d = wid * _FEATS_PER_W + f
            pltpu.async_copy(c_t_hbm.at[d], row_buf, sem).wait()
            outs = [None, None]
            for c in range(_NFCHUNK):
                p = c % 2
                if outs[p] is not None:
                    outs[p].wait()

                def groups(g, carry):
                    vec = idx_all[pl.ds(c * _FCHUNK + g * _LANES, _LANES)]
                    vals = plsc.load_gather(row_buf, [vec])
                    out_bufs[p, pl.ds(g * _LANES, _LANES)] = vals
                    return carry

                lax.fori_loop(0, _FCHUNK // _LANES, groups, 0)
                outs[p] = pltpu.async_copy(
                    out_bufs.at[p],
                    fc_t_hbm.at[d, pl.ds(c * _FCHUNK, _FCHUNK)], osem)
            outs[0].wait()
            outs[1].wait()

    pl.run_scoped(
        body,
        pltpu.VMEM((_NROWS,), jnp.float32),
        pltpu.VMEM((_BATCH,), jnp.int32),
        pltpu.VMEM((2, _FCHUNK), jnp.float32),
        pltpu.SemaphoreType.DMA,
    )


def _wide_phase(a_hbm, s_hbm, idx_hbm, fa_hbm, fs_hbm, wid, sem):
    base = wid * _B_PER_W

    def body(idx_v, wide_bufs, osem):
        pltpu.sync_copy(idx_hbm.at[pl.ds(base, _B_PER_W)], idx_v)
        jobs = []
        for j in range(_NCHUNK):
            jobs.append((a_hbm, fa_hbm, j * _CHUNK))
            jobs.append((s_hbm, fs_hbm, j * _CHUNK))

        def start_gather(k, p):
            tbl, _, off = jobs[k]
            return pltpu.async_copy(
                tbl.at[idx_v.at[pl.ds(off, _CHUNK)]], wide_bufs.at[p], sem)

        gathers = [None] * _NBUF
        outs = [None] * _NBUF
        for k in range(_NBUF):
            gathers[k] = start_gather(k, k)
        njobs = len(jobs)
        for k in range(njobs):
            p = k % _NBUF
            gathers[p].wait()
            _, out_hbm, off = jobs[k]
            outs[p] = pltpu.async_copy(
                wide_bufs.at[p], out_hbm.at[pl.ds(base + off, _CHUNK)], osem)
            if k + _NBUF < njobs:
                outs[p].wait()
                gathers[p] = start_gather(k + _NBUF, p)
                outs[p] = None
        for p in range(_NBUF):
            if outs[p] is not None:
                outs[p].wait()

    pl.run_scoped(
        body,
        pltpu.VMEM((_B_PER_W,), jnp.int32),
        pltpu.VMEM((_NBUF, _CHUNK, 256), jnp.float32),
        pltpu.SemaphoreType.DMA,
    )


def _sc_gather(c_t_hbm, a_hbm, s_hbm, idx_hbm,
               fc_t_hbm, fa_hbm, fs_hbm, sem):
    wid = lax.axis_index("s") * _NC + lax.axis_index("c")
    _fc_phase(c_t_hbm, idx_hbm, fc_t_hbm, wid, sem)
    _wide_phase(a_hbm, s_hbm, idx_hbm, fa_hbm, fs_hbm, wid, sem)


def _fclass_body(fct_ref, w_ref, b_ref, out_ref):
    out_ref[...] = lax.dot_general(
        w_ref[...], fct_ref[...],
        dimension_numbers=(((1,), (0,)), ((), ())),
        preferred_element_type=jnp.float32) + b_ref[...]


@jax.jit
def kernel(c_latent, a_latent, s_latent, W, b, sample_index):
    fa_dim = a_latent.shape[1]
    fs_dim = s_latent.shape[1]
    ncat = W.shape[0]
    idx = sample_index.astype(jnp.int32)
    c_t = c_latent.T

    mesh = plsc.VectorSubcoreMesh(core_axis_name="c", subcore_axis_name="s")
    sc_call = pl.kernel(
        _sc_gather,
        out_type=(
            jax.ShapeDtypeStruct((c_t.shape[0], _BATCH), jnp.float32),
            jax.ShapeDtypeStruct((_BATCH, fa_dim), jnp.float32),
            jax.ShapeDtypeStruct((_BATCH, fs_dim), jnp.float32),
        ),
        mesh=mesh,
        scratch_types=[pltpu.SemaphoreType.DMA],
        compiler_params=pltpu.CompilerParams(needs_layout_passes=False),
    )
    fc_t, fa, fs = sc_call(c_t, a_latent, s_latent, idx)

    fclass_t = pl.pallas_call(
        _fclass_body,
        out_shape=jax.ShapeDtypeStruct((ncat, _BATCH), jnp.float32),
    )(fc_t, W, b.reshape(ncat, 1))

    return (fc_t.T, fa, fs, fclass_t.T)


# speedup vs baseline: 1.6474x; 1.1106x over previous
import jax
import jax.numpy as jnp
from jax import lax
from jax.experimental import pallas as pl
from jax.experimental.pallas import tpu as pltpu
from jax.experimental.pallas import tpu_sc as plsc

_NC = 2
_NS = 16
_NW = _NC * _NS

_BATCH = 16384
_B_PER_W = _BATCH // _NW
_CHUNK = 128
_NCHUNK = _B_PER_W // _CHUNK
_NBUF = 3
_LANES = 16

_NROWS = 100000
_FEATS_PER_W = 2
_FCHUNK = 2048
_NFCHUNK = _BATCH // _FCHUNK
_UNROLL = 8


def _fc_phase(c_t_hbm, idx_hbm, fc_t_hbm, wid, sem):
    def body(row_buf, idx_all, out_bufs, osem):
        pltpu.sync_copy(idx_hbm, idx_all)
        for f in range(_FEATS_PER_W):
            d = wid * _FEATS_PER_W + f
            pltpu.async_copy(c_t_hbm.at[d], row_buf, sem).wait()
            outs = [None, None]
            for c in range(_NFCHUNK):
                p = c % 2
                if outs[p] is not None:
                    outs[p].wait()

                def groups(g8, carry):
                    for u in range(_UNROLL):
                        off = g8 * (_UNROLL * _LANES) + u * _LANES
                        vec = idx_all[pl.ds(c * _FCHUNK + off, _LANES)]
                        vals = plsc.load_gather(row_buf, [vec])
                        out_bufs[p, pl.ds(off, _LANES)] = vals
                    return carry

                lax.fori_loop(0, _FCHUNK // (_LANES * _UNROLL), groups, 0)
                outs[p] = pltpu.async_copy(
                    out_bufs.at[p],
                    fc_t_hbm.at[d, pl.ds(c * _FCHUNK, _FCHUNK)], osem)
            outs[0].wait()
            outs[1].wait()

    pl.run_scoped(
        body,
        pltpu.VMEM((_NROWS,), jnp.float32),
        pltpu.VMEM((_BATCH,), jnp.int32),
        pltpu.VMEM((2, _FCHUNK), jnp.float32),
        pltpu.SemaphoreType.DMA,
    )


def _wide_phase(a_hbm, s_hbm, idx_hbm, fa_hbm, fs_hbm, wid, sem):
    base = wid * _B_PER_W

    def body(idx_v, wide_bufs, osem):
        pltpu.sync_copy(idx_hbm.at[pl.ds(base, _B_PER_W)], idx_v)
        jobs = []
        for j in range(_NCHUNK):
            jobs.append((a_hbm, fa_hbm, j * _CHUNK))
            jobs.append((s_hbm, fs_hbm, j * _CHUNK))

        def start_gather(k, p):
            tbl, _, off = jobs[k]
            return pltpu.async_copy(
                tbl.at[idx_v.at[pl.ds(off, _CHUNK)]], wide_bufs.at[p], sem)

        gathers = [None] * _NBUF
        outs = [None] * _NBUF
        for k in range(_NBUF):
            gathers[k] = start_gather(k, k)
        njobs = len(jobs)
        for k in range(njobs):
            p = k % _NBUF
            gathers[p].wait()
            _, out_hbm, off = jobs[k]
            outs[p] = pltpu.async_copy(
                wide_bufs.at[p], out_hbm.at[pl.ds(base + off, _CHUNK)], osem)
            if k + _NBUF < njobs:
                outs[p].wait()
                gathers[p] = start_gather(k + _NBUF, p)
                outs[p] = None
        for p in range(_NBUF):
            if outs[p] is not None:
                outs[p].wait()

    pl.run_scoped(
        body,
        pltpu.VMEM((_B_PER_W,), jnp.int32),
        pltpu.VMEM((_NBUF, _CHUNK, 256), jnp.float32),
        pltpu.SemaphoreType.DMA,
    )


def _sc_gather(c_t_hbm, a_hbm, s_hbm, idx_hbm,
               fc_t_hbm, fa_hbm, fs_hbm, sem):
    wid = lax.axis_index("s") * _NC + lax.axis_index("c")
    _fc_phase(c_t_hbm, idx_hbm, fc_t_hbm, wid, sem)
    _wide_phase(a_hbm, s_hbm, idx_hbm, fa_hbm, fs_hbm, wid, sem)


def _fclass_body(fct_ref, w_ref, b_ref, out_ref):
    out_ref[...] = lax.dot_general(
        w_ref[...], fct_ref[...],
        dimension_numbers=(((1,), (0,)), ((), ())),
        preferred_element_type=jnp.float32) + b_ref[...]


@jax.jit
def kernel(c_latent, a_latent, s_latent, W, b, sample_index):
    fa_dim = a_latent.shape[1]
    fs_dim = s_latent.shape[1]
    ncat = W.shape[0]
    idx = sample_index.astype(jnp.int32)
    c_t = c_latent.T

    mesh = plsc.VectorSubcoreMesh(core_axis_name="c", subcore_axis_name="s")
    sc_call = pl.kernel(
        _sc_gather,
        out_type=(
            jax.ShapeDtypeStruct((c_t.shape[0], _BATCH), jnp.float32),
            jax.ShapeDtypeStruct((_BATCH, fa_dim), jnp.float32),
            jax.ShapeDtypeStruct((_BATCH, fs_dim), jnp.float32),
        ),
        mesh=mesh,
        scratch_types=[pltpu.SemaphoreType.DMA],
        compiler_params=pltpu.CompilerParams(needs_layout_passes=False),
    )
    fc_t, fa, fs = sc_call(c_t, a_latent, s_latent, idx)

    fclass_t = pl.pallas_call(
        _fclass_body,
        out_shape=jax.ShapeDtypeStruct((ncat, _BATCH), jnp.float32),
    )(fc_t, W, b.reshape(ncat, 1))

    return (fc_t.T, fa, fs, fclass_t.T)


# speedup vs baseline: 1.6585x; 1.0068x over previous
import jax
import jax.numpy as jnp
from jax import lax
from jax.experimental import pallas as pl
from jax.experimental.pallas import tpu as pltpu
from jax.experimental.pallas import tpu_sc as plsc

_NC = 2
_NS = 16
_NW = _NC * _NS

_BATCH = 16384
_B_PER_W = _BATCH // _NW
_CHUNK = 128
_NCHUNK = _B_PER_W // _CHUNK
_NBUF = 3
_LANES = 16

_NROWS = 100000
_FEATS_PER_W = 2
_FCHUNK = 4096
_NFCHUNK = _BATCH // _FCHUNK
_UNROLL = 16


def _fc_phase(c_t_hbm, idx_hbm, fc_t_hbm, wid, sem):
    def body(row_buf, idx_all, out_bufs, osem):
        pltpu.sync_copy(idx_hbm, idx_all)
        for f in range(_FEATS_PER_W):
            d = wid * _FEATS_PER_W + f
            pltpu.async_copy(c_t_hbm.at[d], row_buf, sem).wait()
            outs = [None, None]
            for c in range(_NFCHUNK):
                p = c % 2
                if outs[p] is not None:
                    outs[p].wait()

                def groups(g8, carry):
                    for u in range(_UNROLL):
                        off = g8 * (_UNROLL * _LANES) + u * _LANES
                        vec = idx_all[pl.ds(c * _FCHUNK + off, _LANES)]
                        vals = plsc.load_gather(row_buf, [vec])
                        out_bufs[p, pl.ds(off, _LANES)] = vals
                    return carry

                lax.fori_loop(0, _FCHUNK // (_LANES * _UNROLL), groups, 0)
                outs[p] = pltpu.async_copy(
                    out_bufs.at[p],
                    fc_t_hbm.at[d, pl.ds(c * _FCHUNK, _FCHUNK)], osem)
            outs[0].wait()
            outs[1].wait()

    pl.run_scoped(
        body,
        pltpu.VMEM((_NROWS,), jnp.float32),
        pltpu.VMEM((_BATCH,), jnp.int32),
        pltpu.VMEM((2, _FCHUNK), jnp.float32),
        pltpu.SemaphoreType.DMA,
    )


def _wide_phase(a_hbm, s_hbm, idx_hbm, fa_hbm, fs_hbm, wid, sem):
    base = wid * _B_PER_W

    def body(idx_v, wide_bufs, osem):
        pltpu.sync_copy(idx_hbm.at[pl.ds(base, _B_PER_W)], idx_v)
        jobs = []
        for j in range(_NCHUNK):
            jobs.append((a_hbm, fa_hbm, j * _CHUNK))
            jobs.append((s_hbm, fs_hbm, j * _CHUNK))

        def start_gather(k, p):
            tbl, _, off = jobs[k]
            return pltpu.async_copy(
                tbl.at[idx_v.at[pl.ds(off, _CHUNK)]], wide_bufs.at[p], sem)

        gathers = [None] * _NBUF
        outs = [None] * _NBUF
        for k in range(_NBUF):
            gathers[k] = start_gather(k, k)
        njobs = len(jobs)
        for k in range(njobs):
            p = k % _NBUF
            gathers[p].wait()
            _, out_hbm, off = jobs[k]
            outs[p] = pltpu.async_copy(
                wide_bufs.at[p], out_hbm.at[pl.ds(base + off, _CHUNK)], osem)
            if k + _NBUF < njobs:
                outs[p].wait()
                gathers[p] = start_gather(k + _NBUF, p)
                outs[p] = None
        for p in range(_NBUF):
            if outs[p] is not None:
                outs[p].wait()

    pl.run_scoped(
        body,
        pltpu.VMEM((_B_PER_W,), jnp.int32),
        pltpu.VMEM((_NBUF, _CHUNK, 256), jnp.float32),
        pltpu.SemaphoreType.DMA,
    )


def _sc_gather(c_t_hbm, a_hbm, s_hbm, idx_hbm,
               fc_t_hbm, fa_hbm, fs_hbm, sem):
    wid = lax.axis_index("s") * _NC + lax.axis_index("c")
    _fc_phase(c_t_hbm, idx_hbm, fc_t_hbm, wid, sem)
    _wide_phase(a_hbm, s_hbm, idx_hbm, fa_hbm, fs_hbm, wid, sem)


def _fclass_body(fct_ref, w_ref, b_ref, out_ref):
    out_ref[...] = lax.dot_general(
        w_ref[...], fct_ref[...],
        dimension_numbers=(((1,), (0,)), ((), ())),
        preferred_element_type=jnp.float32) + b_ref[...]


@jax.jit
def kernel(c_latent, a_latent, s_latent, W, b, sample_index):
    fa_dim = a_latent.shape[1]
    fs_dim = s_latent.shape[1]
    ncat = W.shape[0]
    idx = sample_index.astype(jnp.int32)
    c_t = c_latent.T

    mesh = plsc.VectorSubcoreMesh(core_axis_name="c", subcore_axis_name="s")
    sc_call = pl.kernel(
        _sc_gather,
        out_type=(
            jax.ShapeDtypeStruct((c_t.shape[0], _BATCH), jnp.float32),
            jax.ShapeDtypeStruct((_BATCH, fa_dim), jnp.float32),
            jax.ShapeDtypeStruct((_BATCH, fs_dim), jnp.float32),
        ),
        mesh=mesh,
        scratch_types=[pltpu.SemaphoreType.DMA],
        compiler_params=pltpu.CompilerParams(needs_layout_passes=False),
    )
    fc_t, fa, fs = sc_call(c_t, a_latent, s_latent, idx)

    fclass_t = pl.pallas_call(
        _fclass_body,
        out_shape=jax.ShapeDtypeStruct((ncat, _BATCH), jnp.float32),
    )(fc_t, W, b.reshape(ncat, 1))

    return (fc_t.T, fa, fs, fclass_t.T)
